# Initial kernel scaffold; baseline (speedup 1.0000x reference)
import jax
import jax.numpy as jnp
from jax.experimental import pallas as pl


def kernel(char_indices, text_embeddings, char_codebook, W1, b1, W2, b2):
    raise NotImplementedError("write your pallas kernel here")


# speedup vs baseline: 1.6724x; 1.6724x over previous
import functools

import jax
import jax.numpy as jnp
from jax import lax
from jax.experimental import pallas as pl
from jax.experimental.pallas import tpu as pltpu
from jax.experimental.pallas import tpu_sc as plsc

B, T = 4096, 50
NUM_WORDS = T // 2
CB_SIZE, CB_DIM, LLM_DIM = 256, 512, 768
L = 16
T_PAD = 64


def _sc_words_body(idx_hbm, cb_hbm, words_hbm, pooled_hbm,
                   cb_v, idx_v, words_v, pooled_v):
    dhalf = lax.axis_index("c")
    grp = lax.axis_index("s")
    rows_per_grp = B // 16
    dh = CB_DIM // 2
    nchunk = dh // L

    pltpu.sync_copy(cb_hbm.at[:, pl.ds(dhalf * dh, dh)], cb_v)
    base = grp * rows_per_grp
    pltpu.sync_copy(idx_hbm.at[pl.ds(base, rows_per_grp)], idx_v)

    def row_body(r, carry):
        row = base + r
        ivs = [idx_v[r, pl.ds(k * L, L)] for k in range(T_PAD // L)]
        ids = [ivs[t // L][t % L] for t in range(T)]
        for c in range(nchunk):
            sl = pl.ds(c * L, L)
            acc = jnp.zeros((L,), jnp.float32)
            for w in range(NUM_WORDS):
                a = cb_v[ids[2 * w], sl]
                b = cb_v[ids[2 * w + 1], sl]
                s = a + b
                words_v[w, sl] = s * 0.5
                acc = acc + s
            pooled_v[sl] = acc * (1.0 / T)
        pltpu.sync_copy(words_v, words_hbm.at[row, :, pl.ds(dhalf * dh, dh)])
        pltpu.sync_copy(pooled_v, pooled_hbm.at[row, pl.ds(dhalf * dh, dh)])
        return carry

    lax.fori_loop(0, rows_per_grp, row_body, 0)


def _sc_words(char_indices, char_codebook):
    mesh = plsc.VectorSubcoreMesh(core_axis_name="c", subcore_axis_name="s")
    rows_per_grp = B // 16
    dh = CB_DIM // 2
    f = pl.kernel(
        _sc_words_body,
        out_type=[
            jax.ShapeDtypeStruct((B, NUM_WORDS, CB_DIM), jnp.float32),
            jax.ShapeDtypeStruct((B, CB_DIM), jnp.float32),
        ],
        mesh=mesh,
        scratch_types=[
            pltpu.VMEM((CB_SIZE, dh), jnp.float32),
            pltpu.VMEM((rows_per_grp, T_PAD), jnp.int32),
            pltpu.VMEM((NUM_WORDS, dh), jnp.float32),
            pltpu.VMEM((dh,), jnp.float32),
        ],
    )
    return f(char_indices, char_codebook)


def _tc_loss_body(pooled_ref, text_ref, w1_ref, b1_ref, w2_ref, b2_ref,
                  out_ref):
    pooled = pooled_ref[...]
    h = jnp.dot(pooled, w1_ref[...], preferred_element_type=jnp.float32)
    h = jnp.maximum(h + b1_ref[...][None, :], 0.0)
    hbar = jnp.mean(h, axis=0, keepdims=True)
    proj = jnp.dot(hbar, w2_ref[...], preferred_element_type=jnp.float32)
    proj = proj + b2_ref[...][None, :]
    tbar = jnp.mean(text_ref[...], axis=0, keepdims=True)
    d = proj - tbar
    out_ref[...] = jnp.reshape(jnp.mean(d * d), (1, 1))


def _tc_loss(pooled, text_embeddings, W1, b1, W2, b2):
    return pl.pallas_call(
        _tc_loss_body,
        out_shape=jax.ShapeDtypeStruct((1, 1), jnp.float32),
    )(pooled, text_embeddings, W1, b1, W2, b2)


def kernel(char_indices, text_embeddings, char_codebook, W1, b1, W2, b2):
    idx = jnp.pad(char_indices.astype(jnp.int32), ((0, 0), (0, T_PAD - T)))
    words, pooled = _sc_words(idx, char_codebook)
    loss = _tc_loss(pooled, text_embeddings, W1, b1, W2, b2)
    return words, loss[0, 0]


# speedup vs baseline: 1.6810x; 1.0052x over previous
import functools

import jax
import jax.numpy as jnp
from jax import lax
from jax.experimental import pallas as pl
from jax.experimental.pallas import tpu as pltpu
from jax.experimental.pallas import tpu_sc as plsc

B, T = 4096, 50
NUM_WORDS = T // 2
CB_SIZE, CB_DIM, LLM_DIM = 256, 512, 768
L = 16
T_PAD = 64


def _sc_words_body(idx_hbm, cb_hbm, words_hbm, pooled_hbm,
                   cb_v, idx_v, words_v, pooled_v, sem0, sem1):
    dhalf = lax.axis_index("c")
    grp = lax.axis_index("s")
    rows_per_grp = B // 16
    dh = CB_DIM // 2
    nchunk = dh // L

    pltpu.sync_copy(cb_hbm.at[:, pl.ds(dhalf * dh, dh)], cb_v)
    base = grp * rows_per_grp
    pltpu.sync_copy(idx_hbm.at[pl.ds(base, rows_per_grp)], idx_v)

    def one_row(r, words_b, pooled_b, sem, first):
        row = base + r
        @pl.when(jnp.logical_not(first))
        def _():
            pltpu.make_async_copy(
                words_b, words_hbm.at[row, :, pl.ds(dhalf * dh, dh)], sem
            ).wait()
            pltpu.make_async_copy(
                pooled_b, pooled_hbm.at[row, pl.ds(dhalf * dh, dh)], sem
            ).wait()
        ivs = [idx_v[r, pl.ds(k * L, L)] for k in range(T_PAD // L)]
        ids = [ivs[t // L][t % L] for t in range(T)]
        for c in range(nchunk):
            sl = pl.ds(c * L, L)
            acc = jnp.zeros((L,), jnp.float32)
            for w in range(NUM_WORDS):
                a = cb_v[ids[2 * w], sl]
                b = cb_v[ids[2 * w + 1], sl]
                s = a + b
                words_b[w, sl] = s * 0.5
                acc = acc + s
            pooled_b[sl] = acc * (1.0 / T)
        pltpu.async_copy(
            words_b, words_hbm.at[row, :, pl.ds(dhalf * dh, dh)], sem)
        pltpu.async_copy(
            pooled_b, pooled_hbm.at[row, pl.ds(dhalf * dh, dh)], sem)

    def pair_body(i, carry):
        one_row(2 * i, words_v.at[0], pooled_v.at[0], sem0, i == 0)
        one_row(2 * i + 1, words_v.at[1], pooled_v.at[1], sem1, i == 0)
        return carry

    lax.fori_loop(0, rows_per_grp // 2, pair_body, 0)
    last = base + rows_per_grp - 2
    pltpu.make_async_copy(
        words_v.at[0], words_hbm.at[last, :, pl.ds(dhalf * dh, dh)], sem0
    ).wait()
    pltpu.make_async_copy(
        pooled_v.at[0], pooled_hbm.at[last, pl.ds(dhalf * dh, dh)], sem0
    ).wait()
    pltpu.make_async_copy(
        words_v.at[1], words_hbm.at[last + 1, :, pl.ds(dhalf * dh, dh)], sem1
    ).wait()
    pltpu.make_async_copy(
        pooled_v.at[1], pooled_hbm.at[last + 1, pl.ds(dhalf * dh, dh)], sem1
    ).wait()


def _sc_words(char_indices, char_codebook):
    mesh = plsc.VectorSubcoreMesh(core_axis_name="c", subcore_axis_name="s")
    rows_per_grp = B // 16
    dh = CB_DIM // 2
    f = pl.kernel(
        _sc_words_body,
        out_type=[
            jax.ShapeDtypeStruct((B, NUM_WORDS, CB_DIM), jnp.float32),
            jax.ShapeDtypeStruct((B, CB_DIM), jnp.float32),
        ],
        mesh=mesh,
        scratch_types=[
            pltpu.VMEM((CB_SIZE, dh), jnp.float32),
            pltpu.VMEM((rows_per_grp, T_PAD), jnp.int32),
            pltpu.VMEM((2, NUM_WORDS, dh), jnp.float32),
            pltpu.VMEM((2, dh), jnp.float32),
            pltpu.SemaphoreType.DMA,
            pltpu.SemaphoreType.DMA,
        ],
    )
    return f(char_indices, char_codebook)


def _tc_loss_body(pooled_ref, text_ref, w1_ref, b1_ref, w2_ref, b2_ref,
                  out_ref):
    pooled = pooled_ref[...]
    h = jnp.dot(pooled, w1_ref[...], preferred_element_type=jnp.float32)
    h = jnp.maximum(h + b1_ref[...][None, :], 0.0)
    hbar = jnp.mean(h, axis=0, keepdims=True)
    proj = jnp.dot(hbar, w2_ref[...], preferred_element_type=jnp.float32)
    proj = proj + b2_ref[...][None, :]
    tbar = jnp.mean(text_ref[...], axis=0, keepdims=True)
    d = proj - tbar
    out_ref[...] = jnp.reshape(jnp.mean(d * d), (1, 1))


def _tc_loss(pooled, text_embeddings, W1, b1, W2, b2):
    return pl.pallas_call(
        _tc_loss_body,
        out_shape=jax.ShapeDtypeStruct((1, 1), jnp.float32),
    )(pooled, text_embeddings, W1, b1, W2, b2)


def kernel(char_indices, text_embeddings, char_codebook, W1, b1, W2, b2):
    idx = jnp.pad(char_indices.astype(jnp.int32), ((0, 0), (0, T_PAD - T)))
    words, pooled = _sc_words(idx, char_codebook)
    loss = _tc_loss(pooled, text_embeddings, W1, b1, W2, b2)
    return words, loss[0, 0]


# speedup vs baseline: 7.7117x; 4.5875x over previous
import functools

import jax
import jax.numpy as jnp
from jax import lax
from jax.experimental import pallas as pl
from jax.experimental.pallas import tpu as pltpu
from jax.experimental.pallas import tpu_sc as plsc

B, T = 4096, 50
NUM_WORDS = T // 2
CB_SIZE, CB_DIM, LLM_DIM = 256, 512, 768
L = 16
T_PAD = 64


def _sc_words_body(idx_hbm, cb_hbm, words_hbm, pooled_hbm,
                   cb_v, idx_v, words_v, pooled_v, sem0, sem1):
    dhalf = lax.axis_index("c")
    grp = lax.axis_index("s")
    rows_per_grp = B // 16
    dh = CB_DIM // 2
    nchunk = dh // L

    pltpu.sync_copy(cb_hbm.at[:, pl.ds(dhalf * dh, dh)], cb_v)
    base = grp * rows_per_grp
    pltpu.sync_copy(idx_hbm.at[pl.ds(base, rows_per_grp)], idx_v)

    def one_row(r, words_b, pooled_b, sem, first):
        row = base + r
        @pl.when(jnp.logical_not(first))
        def _():
            pltpu.make_async_copy(
                words_b, words_hbm.at[row, :, pl.ds(dhalf * dh, dh)], sem
            ).wait()
            pltpu.make_async_copy(
                pooled_b, pooled_hbm.at[row, pl.ds(dhalf * dh, dh)], sem
            ).wait()
        ivs = [idx_v[r, pl.ds(k * L, L)] for k in range(T_PAD // L)]
        ids = [ivs[t // L][t % L] for t in range(T)]

        @functools.partial(plsc.parallel_loop, 0, nchunk)
        def _(c):
            sl = pl.ds(c * L, L)
            acc = jnp.zeros((L,), jnp.float32)
            for w in range(NUM_WORDS):
                a = cb_v[ids[2 * w], sl]
                b = cb_v[ids[2 * w + 1], sl]
                s = a + b
                words_b[w, sl] = s * 0.5
                acc = acc + s
            pooled_b[sl] = acc * (1.0 / T)
        pltpu.async_copy(
            words_b, words_hbm.at[row, :, pl.ds(dhalf * dh, dh)], sem)
        pltpu.async_copy(
            pooled_b, pooled_hbm.at[row, pl.ds(dhalf * dh, dh)], sem)

    def pair_body(i, carry):
        one_row(2 * i, words_v.at[0], pooled_v.at[0], sem0, i == 0)
        one_row(2 * i + 1, words_v.at[1], pooled_v.at[1], sem1, i == 0)
        return carry

    lax.fori_loop(0, rows_per_grp // 2, pair_body, 0)
    last = base + rows_per_grp - 2
    pltpu.make_async_copy(
        words_v.at[0], words_hbm.at[last, :, pl.ds(dhalf * dh, dh)], sem0
    ).wait()
    pltpu.make_async_copy(
        pooled_v.at[0], pooled_hbm.at[last, pl.ds(dhalf * dh, dh)], sem0
    ).wait()
    pltpu.make_async_copy(
        words_v.at[1], words_hbm.at[last + 1, :, pl.ds(dhalf * dh, dh)], sem1
    ).wait()
    pltpu.make_async_copy(
        pooled_v.at[1], pooled_hbm.at[last + 1, pl.ds(dhalf * dh, dh)], sem1
    ).wait()


def _sc_words(char_indices, char_codebook):
    mesh = plsc.VectorSubcoreMesh(core_axis_name="c", subcore_axis_name="s")
    rows_per_grp = B // 16
    dh = CB_DIM // 2
    f = pl.kernel(
        _sc_words_body,
        out_type=[
            jax.ShapeDtypeStruct((B, NUM_WORDS, CB_DIM), jnp.float32),
            jax.ShapeDtypeStruct((B, CB_DIM), jnp.float32),
        ],
        mesh=mesh,
        scratch_types=[
            pltpu.VMEM((CB_SIZE, dh), jnp.float32),
            pltpu.VMEM((rows_per_grp, T_PAD), jnp.int32),
            pltpu.VMEM((2, NUM_WORDS, dh), jnp.float32),
            pltpu.VMEM((2, dh), jnp.float32),
            pltpu.SemaphoreType.DMA,
            pltpu.SemaphoreType.DMA,
        ],
    )
    return f(char_indices, char_codebook)


def _tc_loss_body(pooled_ref, text_ref, w1_ref, b1_ref, w2_ref, b2_ref,
                  out_ref):
    pooled = pooled_ref[...]
    h = jnp.dot(pooled, w1_ref[...], preferred_element_type=jnp.float32)
    h = jnp.maximum(h + b1_ref[...][None, :], 0.0)
    hbar = jnp.mean(h, axis=0, keepdims=True)
    proj = jnp.dot(hbar, w2_ref[...], preferred_element_type=jnp.float32)
    proj = proj + b2_ref[...][None, :]
    tbar = jnp.mean(text_ref[...], axis=0, keepdims=True)
    d = proj - tbar
    out_ref[...] = jnp.reshape(jnp.mean(d * d), (1, 1))


def _tc_loss(pooled, text_embeddings, W1, b1, W2, b2):
    return pl.pallas_call(
        _tc_loss_body,
        out_shape=jax.ShapeDtypeStruct((1, 1), jnp.float32),
    )(pooled, text_embeddings, W1, b1, W2, b2)


def kernel(char_indices, text_embeddings, char_codebook, W1, b1, W2, b2):
    idx = jnp.pad(char_indices.astype(jnp.int32), ((0, 0), (0, T_PAD - T)))
    words, pooled = _sc_words(idx, char_codebook)
    loss = _tc_loss(pooled, text_embeddings, W1, b1, W2, b2)
    return words, loss[0, 0]
